# baseline (speedup 1.0000x reference)
import functools

import jax
import jax.numpy as jnp
from jax import lax
from jax.experimental import pallas as pl
from jax.experimental.pallas import tpu as pltpu
from jax.experimental.pallas import tpu_sc as plsc

L = 16
NC, NS = 2, 16
NW = NC * NS
B, H, W, C = 8, 512, 512, 3
K = 64
PIX = H * W
PIX_PER_W = (B * PIX) // NW
CHUNK_PIX = 8192
NCHUNK = PIX_PER_W // CHUNK_PIX
GROUPS = CHUNK_PIX // L
TAB_ROWS = K * C + 1


def _sc_body(img_hbm, tab_hbm, out_hbm,
             in_r, in_g, in_b, out_r, out_g, out_b, pal):
    wid = lax.axis_index("s") * NC + lax.axis_index("c")
    batch = wid // 4
    quarter = wid % 4

    pltpu.sync_copy(tab_hbm.at[batch], pal)
    gam = pal[K * C]

    def group_body(g, _):
        sl = pl.ds(g * L, L)
        r = in_r[sl]
        gr = in_g[sl]
        bl = in_b[sl]
        s = jnp.zeros((L,), jnp.float32)
        ar = jnp.zeros((L,), jnp.float32)
        ag = jnp.zeros((L,), jnp.float32)
        ab = jnp.zeros((L,), jnp.float32)
        for k in range(K):
            pr = pal[3 * k]
            pg = pal[3 * k + 1]
            pb = pal[3 * k + 2]
            dr = r - pr
            dg = gr - pg
            db = bl - pb
            d = dr * dr + dg * dg + db * db
            e = jnp.exp(d * gam)
            s = s + e
            ar = ar + e * pr
            ag = ag + e * pg
            ab = ab + e * pb
        inv = 1.0 / s
        out_r[sl] = ar * inv
        out_g[sl] = ag * inv
        out_b[sl] = ab * inv
        return 0

    def chunk_body(ci, _):
        base = batch * (C * PIX) + quarter * PIX_PER_W + ci * CHUNK_PIX
        offs = [pl.multiple_of(base + c * PIX, 8) for c in range(C)]
        pltpu.sync_copy(img_hbm.at[pl.ds(offs[0], CHUNK_PIX)], in_r)
        pltpu.sync_copy(img_hbm.at[pl.ds(offs[1], CHUNK_PIX)], in_g)
        pltpu.sync_copy(img_hbm.at[pl.ds(offs[2], CHUNK_PIX)], in_b)
        lax.fori_loop(0, GROUPS, group_body, 0)
        pltpu.sync_copy(out_r, out_hbm.at[pl.ds(offs[0], CHUNK_PIX)])
        pltpu.sync_copy(out_g, out_hbm.at[pl.ds(offs[1], CHUNK_PIX)])
        pltpu.sync_copy(out_b, out_hbm.at[pl.ds(offs[2], CHUNK_PIX)])
        return 0

    lax.fori_loop(0, NCHUNK, chunk_body, 0)


_sc_quantize = functools.partial(
    pl.kernel,
    out_type=jax.ShapeDtypeStruct((B * C * PIX,), jnp.float32),
    mesh=plsc.VectorSubcoreMesh(core_axis_name="c", subcore_axis_name="s"),
    scratch_types=[
        pltpu.VMEM((CHUNK_PIX,), jnp.float32),
        pltpu.VMEM((CHUNK_PIX,), jnp.float32),
        pltpu.VMEM((CHUNK_PIX,), jnp.float32),
        pltpu.VMEM((CHUNK_PIX,), jnp.float32),
        pltpu.VMEM((CHUNK_PIX,), jnp.float32),
        pltpu.VMEM((CHUNK_PIX,), jnp.float32),
        pltpu.VMEM((TAB_ROWS, L), jnp.float32),
    ],
    compiler_params=pltpu.CompilerParams(needs_layout_passes=False),
)(_sc_body)


@jax.jit
def kernel(images, palettes, temperature):
    planar = jnp.transpose(images, (0, 3, 1, 2))
    flat = planar.reshape(-1)
    scale = (-1.0 / temperature).astype(jnp.float32)
    tab = jnp.concatenate(
        [palettes.reshape(B, K * C),
         jnp.broadcast_to(scale, (B, 1))], axis=1)
    tab16 = jnp.broadcast_to(tab[:, :, None], (B, TAB_ROWS, L))
    out_flat = _sc_quantize(flat, tab16)
    return jnp.transpose(out_flat.reshape(B, C, H, W), (0, 2, 3, 1))


# speedup vs baseline: 2.2013x; 2.2013x over previous
import functools

import jax
import jax.numpy as jnp
from jax import lax
from jax.experimental import pallas as pl
from jax.experimental.pallas import tpu as pltpu
from jax.experimental.pallas import tpu_sc as plsc

L = 16
NC, NS = 2, 16
NW = NC * NS
B, H, W, C = 8, 512, 512, 3
K = 64
PIX = H * W
HS_SC = 256
QPIX = HS_SC * W // 4
CHUNK_PIX = 8192
NCHUNK = QPIX // CHUNK_PIX
GROUPS = CHUNK_PIX // L
NG = 4
TAB_ROWS = 4 * K + 2
ROWB = 16


def _sc_body(img_hbm, tab_hbm, out_hbm,
             in_r, in_g, in_b, out_r, out_g, out_b, pal):
    wid = lax.axis_index("s") * NC + lax.axis_index("c")
    batch = wid // 4
    quarter = wid % 4

    pltpu.sync_copy(tab_hbm.at[batch], pal)
    hrec = pal[4 * K + 1]

    def group_body(g, _):
        sls = [pl.ds(g * NG * L + j * L, L) for j in range(NG)]
        r = [in_r[sl] for sl in sls]
        gr = [in_g[sl] for sl in sls]
        bl = [in_b[sl] for sl in sls]
        z = jnp.zeros((L,), jnp.float32)
        s, ar, ag, ab = [z] * NG, [z] * NG, [z] * NG, [z] * NG
        for k in range(K):
            aa = pal[4 * k]
            cr = pal[4 * k + 1]
            cg = pal[4 * k + 2]
            cb = pal[4 * k + 3]
            for j in range(NG):
                t = (aa + cr * r[j]) + cg * gr[j] + cb * bl[j]
                e = jnp.exp(t)
                s[j] = s[j] + e
                ar[j] = ar[j] + e * cr
                ag[j] = ag[j] + e * cg
                ab[j] = ab[j] + e * cb
        for j in range(NG):
            inv = hrec / s[j]
            out_r[sls[j]] = ar[j] * inv
            out_g[sls[j]] = ag[j] * inv
            out_b[sls[j]] = ab[j] * inv
        return 0

    def chunk_body(ci, _):
        pix = quarter * QPIX + ci * CHUNK_PIX
        i_offs = [pl.multiple_of(batch * (C * PIX) + c * PIX + pix, 8)
                  for c in range(C)]
        o_offs = [pl.multiple_of(
            batch * (C * HS_SC * W) + c * (HS_SC * W) + pix, 8)
            for c in range(C)]
        pltpu.sync_copy(img_hbm.at[pl.ds(i_offs[0], CHUNK_PIX)], in_r)
        pltpu.sync_copy(img_hbm.at[pl.ds(i_offs[1], CHUNK_PIX)], in_g)
        pltpu.sync_copy(img_hbm.at[pl.ds(i_offs[2], CHUNK_PIX)], in_b)
        lax.fori_loop(0, GROUPS // NG, group_body, 0)
        pltpu.sync_copy(out_r, out_hbm.at[pl.ds(o_offs[0], CHUNK_PIX)])
        pltpu.sync_copy(out_g, out_hbm.at[pl.ds(o_offs[1], CHUNK_PIX)])
        pltpu.sync_copy(out_b, out_hbm.at[pl.ds(o_offs[2], CHUNK_PIX)])
        return 0

    lax.fori_loop(0, NCHUNK, chunk_body, 0)


_sc_quantize = functools.partial(
    pl.kernel,
    out_type=jax.ShapeDtypeStruct((B * C * HS_SC * W,), jnp.float32),
    mesh=plsc.VectorSubcoreMesh(core_axis_name="c", subcore_axis_name="s"),
    scratch_types=[
        pltpu.VMEM((CHUNK_PIX,), jnp.float32),
        pltpu.VMEM((CHUNK_PIX,), jnp.float32),
        pltpu.VMEM((CHUNK_PIX,), jnp.float32),
        pltpu.VMEM((CHUNK_PIX,), jnp.float32),
        pltpu.VMEM((CHUNK_PIX,), jnp.float32),
        pltpu.VMEM((CHUNK_PIX,), jnp.float32),
        pltpu.VMEM((TAB_ROWS, L), jnp.float32),
    ],
    compiler_params=pltpu.CompilerParams(needs_layout_passes=False),
)(_sc_body)


def _tc_body(img_ref, tab_ref, out_ref):
    r = img_ref[0, 0]
    g = img_ref[0, 1]
    b = img_ref[0, 2]
    z = jnp.zeros((ROWB, W), jnp.float32)
    s, ar, ag, ab = z, z, z, z
    for k in range(K):
        aa = tab_ref[0, 0, 4 * k]
        cr = tab_ref[0, 0, 4 * k + 1]
        cg = tab_ref[0, 0, 4 * k + 2]
        cb = tab_ref[0, 0, 4 * k + 3]
        t = (aa + cr * r) + cg * g + cb * b
        e = jnp.exp(t)
        s = s + e
        ar = ar + e * cr
        ag = ag + e * cg
        ab = ab + e * cb
    inv = tab_ref[0, 0, 4 * K + 1] / s
    out_ref[0, 0] = ar * inv
    out_ref[0, 1] = ag * inv
    out_ref[0, 2] = ab * inv


_tc_quantize = pl.pallas_call(
    _tc_body,
    grid=(B, (H - HS_SC) // ROWB),
    in_specs=[
        pl.BlockSpec((1, C, ROWB, W),
                     lambda b, rb: (b, 0, HS_SC // ROWB + rb, 0)),
        pl.BlockSpec((1, 1, TAB_ROWS), lambda b, rb: (b, 0, 0)),
    ],
    out_specs=pl.BlockSpec((1, C, ROWB, W), lambda b, rb: (b, 0, rb, 0)),
    out_shape=jax.ShapeDtypeStruct((B, C, H - HS_SC, W), jnp.float32),
)


@jax.jit
def kernel(images, palettes, temperature):
    planar = jnp.transpose(images, (0, 3, 1, 2))
    flat = planar.reshape(-1)
    g2 = (-1.0 / temperature).astype(jnp.float32)
    aa = g2 * jnp.sum(palettes * palettes, axis=-1, keepdims=True)
    cc = (-2.0 * g2) * palettes
    per_k = jnp.concatenate([aa, cc], axis=-1).reshape(B, 4 * K)
    extra = jnp.stack([jnp.broadcast_to(g2, (B,)),
                       jnp.broadcast_to(-0.5 / g2, (B,))], axis=1)
    tab = jnp.concatenate([per_k, extra], axis=1)
    tab16 = jnp.broadcast_to(tab[:, :, None], (B, TAB_ROWS, L))
    sc_flat = _sc_quantize(flat, tab16)
    tc_part = _tc_quantize(planar, tab[:, None, :])
    sc_part = sc_flat.reshape(B, C, HS_SC, W)
    full = jnp.concatenate([sc_part, tc_part], axis=2)
    return jnp.transpose(full, (0, 2, 3, 1))


# speedup vs baseline: 3.4249x; 1.5559x over previous
import functools

import jax
import jax.numpy as jnp
from jax import lax
from jax.experimental import pallas as pl
from jax.experimental.pallas import tpu as pltpu
from jax.experimental.pallas import tpu_sc as plsc

L = 16
NC, NS = 2, 16
NW = NC * NS
B, H, W, C = 8, 512, 512, 3
K = 64
PIX = H * W
HS_SC = 128
QPIX = HS_SC * W // 4
CHUNK_PIX = 8192
NCHUNK = QPIX // CHUNK_PIX
GROUPS = CHUNK_PIX // L
NG = 4
TAB_ROWS = 4 * K + 2
ROWB = 32


def _sc_body(img_hbm, tab_hbm, out_hbm,
             in_r, in_g, in_b, out_r, out_g, out_b, pal):
    wid = lax.axis_index("s") * NC + lax.axis_index("c")
    batch = wid // 4
    quarter = wid % 4

    pltpu.sync_copy(tab_hbm.at[batch], pal)
    hrec = pal[4 * K + 1]

    def group_body(g, _):
        sls = [pl.ds(g * NG * L + j * L, L) for j in range(NG)]
        r = [in_r[sl] for sl in sls]
        gr = [in_g[sl] for sl in sls]
        bl = [in_b[sl] for sl in sls]
        z = jnp.zeros((L,), jnp.float32)
        s, ar, ag, ab = [z] * NG, [z] * NG, [z] * NG, [z] * NG
        for k in range(K):
            aa = pal[4 * k]
            cr = pal[4 * k + 1]
            cg = pal[4 * k + 2]
            cb = pal[4 * k + 3]
            for j in range(NG):
                t = (aa + cr * r[j]) + cg * gr[j] + cb * bl[j]
                e = jnp.exp(t)
                s[j] = s[j] + e
                ar[j] = ar[j] + e * cr
                ag[j] = ag[j] + e * cg
                ab[j] = ab[j] + e * cb
        for j in range(NG):
            inv = hrec / s[j]
            out_r[sls[j]] = ar[j] * inv
            out_g[sls[j]] = ag[j] * inv
            out_b[sls[j]] = ab[j] * inv
        return 0

    def chunk_body(ci, _):
        pix = quarter * QPIX + ci * CHUNK_PIX
        i_offs = [pl.multiple_of(batch * (C * PIX) + c * PIX + pix, 8)
                  for c in range(C)]
        o_offs = [pl.multiple_of(
            batch * (C * HS_SC * W) + c * (HS_SC * W) + pix, 8)
            for c in range(C)]
        pltpu.sync_copy(img_hbm.at[pl.ds(i_offs[0], CHUNK_PIX)], in_r)
        pltpu.sync_copy(img_hbm.at[pl.ds(i_offs[1], CHUNK_PIX)], in_g)
        pltpu.sync_copy(img_hbm.at[pl.ds(i_offs[2], CHUNK_PIX)], in_b)
        lax.fori_loop(0, GROUPS // NG, group_body, 0)
        pltpu.sync_copy(out_r, out_hbm.at[pl.ds(o_offs[0], CHUNK_PIX)])
        pltpu.sync_copy(out_g, out_hbm.at[pl.ds(o_offs[1], CHUNK_PIX)])
        pltpu.sync_copy(out_b, out_hbm.at[pl.ds(o_offs[2], CHUNK_PIX)])
        return 0

    lax.fori_loop(0, NCHUNK, chunk_body, 0)


_sc_quantize = functools.partial(
    pl.kernel,
    out_type=jax.ShapeDtypeStruct((B * C * HS_SC * W,), jnp.float32),
    mesh=plsc.VectorSubcoreMesh(core_axis_name="c", subcore_axis_name="s"),
    scratch_types=[
        pltpu.VMEM((CHUNK_PIX,), jnp.float32),
        pltpu.VMEM((CHUNK_PIX,), jnp.float32),
        pltpu.VMEM((CHUNK_PIX,), jnp.float32),
        pltpu.VMEM((CHUNK_PIX,), jnp.float32),
        pltpu.VMEM((CHUNK_PIX,), jnp.float32),
        pltpu.VMEM((CHUNK_PIX,), jnp.float32),
        pltpu.VMEM((TAB_ROWS, L), jnp.float32),
    ],
    compiler_params=pltpu.CompilerParams(needs_layout_passes=False),
)(_sc_body)


def _tc_body(img_ref, tab_ref, out_ref):
    r = img_ref[0, 0]
    g = img_ref[0, 1]
    b = img_ref[0, 2]
    z = jnp.zeros((ROWB, W), jnp.float32)
    s, ar, ag, ab = z, z, z, z
    for k in range(K):
        aa = tab_ref[0, 0, 4 * k]
        cr = tab_ref[0, 0, 4 * k + 1]
        cg = tab_ref[0, 0, 4 * k + 2]
        cb = tab_ref[0, 0, 4 * k + 3]
        t = (aa + cr * r) + cg * g + cb * b
        e = jnp.exp(t)
        s = s + e
        ar = ar + e * cr
        ag = ag + e * cg
        ab = ab + e * cb
    inv = tab_ref[0, 0, 4 * K + 1] / s
    out_ref[0, 0] = ar * inv
    out_ref[0, 1] = ag * inv
    out_ref[0, 2] = ab * inv


_tc_quantize = pl.pallas_call(
    _tc_body,
    grid=(B, (H - HS_SC) // ROWB),
    in_specs=[
        pl.BlockSpec((1, C, ROWB, W),
                     lambda b, rb: (b, 0, HS_SC // ROWB + rb, 0)),
        pl.BlockSpec((1, 1, TAB_ROWS), lambda b, rb: (b, 0, 0)),
    ],
    out_specs=pl.BlockSpec((1, C, ROWB, W), lambda b, rb: (b, 0, rb, 0)),
    out_shape=jax.ShapeDtypeStruct((B, C, H - HS_SC, W), jnp.float32),
)


@jax.jit
def kernel(images, palettes, temperature):
    planar = jnp.transpose(images, (0, 3, 1, 2))
    flat = planar.reshape(-1)
    g2 = (-1.0 / temperature).astype(jnp.float32)
    aa = g2 * jnp.sum(palettes * palettes, axis=-1, keepdims=True)
    cc = (-2.0 * g2) * palettes
    per_k = jnp.concatenate([aa, cc], axis=-1).reshape(B, 4 * K)
    extra = jnp.stack([jnp.broadcast_to(g2, (B,)),
                       jnp.broadcast_to(-0.5 / g2, (B,))], axis=1)
    tab = jnp.concatenate([per_k, extra], axis=1)
    tab16 = jnp.broadcast_to(tab[:, :, None], (B, TAB_ROWS, L))
    sc_flat = _sc_quantize(flat, tab16)
    tc_part = _tc_quantize(planar, tab[:, None, :])
    sc_part = sc_flat.reshape(B, C, HS_SC, W)
    full = jnp.concatenate([sc_part, tc_part], axis=2)
    return jnp.transpose(full, (0, 2, 3, 1))


# speedup vs baseline: 3.5564x; 1.0384x over previous
import functools

import jax
import jax.numpy as jnp
from jax import lax
from jax.experimental import pallas as pl
from jax.experimental.pallas import tpu as pltpu
from jax.experimental.pallas import tpu_sc as plsc

L = 16
NC, NS = 2, 16
NW = NC * NS
B, H, W, C = 8, 512, 512, 3
K = 64
PIX = H * W
HS_SC = 144
QPIX = HS_SC * W // 4
CHUNK_PIX = 2048
NCHUNK = QPIX // CHUNK_PIX
GROUPS = CHUNK_PIX // L
NG = 4
TAB_ROWS = 4 * K + 8
ROWB = 16


def _sc_body(img_hbm, tab_hbm, out_hbm,
             in_r, in_g, in_b, out_r, out_g, out_b, pal_s, pal):
    wid = lax.axis_index("s") * NC + lax.axis_index("c")
    batch = wid // 4
    quarter = wid % 4

    pltpu.sync_copy(tab_hbm.at[pl.ds(batch * TAB_ROWS, TAB_ROWS)], pal_s)

    def splat_body(i, _):
        pal[i] = plsc.load_gather(pal_s, [jnp.full((L,), i, jnp.int32)])
        return 0

    lax.fori_loop(0, TAB_ROWS, splat_body, 0)
    hrec = pal[4 * K + 1]

    def group_body(g, _):
        sls = [pl.ds(g * NG * L + j * L, L) for j in range(NG)]
        r = [in_r[sl] for sl in sls]
        gr = [in_g[sl] for sl in sls]
        bl = [in_b[sl] for sl in sls]
        z = jnp.zeros((L,), jnp.float32)
        s, ar, ag, ab = [z] * NG, [z] * NG, [z] * NG, [z] * NG
        for k in range(K):
            aa = pal[4 * k]
            cr = pal[4 * k + 1]
            cg = pal[4 * k + 2]
            cb = pal[4 * k + 3]
            for j in range(NG):
                t = (aa + cr * r[j]) + cg * gr[j] + cb * bl[j]
                e = jnp.exp(t)
                s[j] = s[j] + e
                ar[j] = ar[j] + e * cr
                ag[j] = ag[j] + e * cg
                ab[j] = ab[j] + e * cb
        for j in range(NG):
            inv = hrec / s[j]
            out_r[sls[j]] = ar[j] * inv
            out_g[sls[j]] = ag[j] * inv
            out_b[sls[j]] = ab[j] * inv
        return 0

    def chunk_body(ci, _):
        pix = quarter * QPIX + ci * CHUNK_PIX
        i_offs = [pl.multiple_of(batch * (C * PIX) + c * PIX + pix, 8)
                  for c in range(C)]
        o_offs = [pl.multiple_of(
            batch * (C * HS_SC * W) + c * (HS_SC * W) + pix, 8)
            for c in range(C)]
        pltpu.sync_copy(img_hbm.at[pl.ds(i_offs[0], CHUNK_PIX)], in_r)
        pltpu.sync_copy(img_hbm.at[pl.ds(i_offs[1], CHUNK_PIX)], in_g)
        pltpu.sync_copy(img_hbm.at[pl.ds(i_offs[2], CHUNK_PIX)], in_b)
        lax.fori_loop(0, GROUPS // NG, group_body, 0)
        pltpu.sync_copy(out_r, out_hbm.at[pl.ds(o_offs[0], CHUNK_PIX)])
        pltpu.sync_copy(out_g, out_hbm.at[pl.ds(o_offs[1], CHUNK_PIX)])
        pltpu.sync_copy(out_b, out_hbm.at[pl.ds(o_offs[2], CHUNK_PIX)])
        return 0

    lax.fori_loop(0, NCHUNK, chunk_body, 0)


_sc_quantize = functools.partial(
    pl.kernel,
    out_type=jax.ShapeDtypeStruct((B * C * HS_SC * W,), jnp.float32),
    mesh=plsc.VectorSubcoreMesh(core_axis_name="c", subcore_axis_name="s"),
    scratch_types=[
        pltpu.VMEM((CHUNK_PIX,), jnp.float32),
        pltpu.VMEM((CHUNK_PIX,), jnp.float32),
        pltpu.VMEM((CHUNK_PIX,), jnp.float32),
        pltpu.VMEM((CHUNK_PIX,), jnp.float32),
        pltpu.VMEM((CHUNK_PIX,), jnp.float32),
        pltpu.VMEM((CHUNK_PIX,), jnp.float32),
        pltpu.VMEM((TAB_ROWS,), jnp.float32),
        pltpu.VMEM((TAB_ROWS, L), jnp.float32),
    ],
    compiler_params=pltpu.CompilerParams(needs_layout_passes=False),
)(_sc_body)


def _tc_body(img_ref, tab_ref, out_ref):
    r = img_ref[0, 0]
    g = img_ref[0, 1]
    b = img_ref[0, 2]
    z = jnp.zeros((ROWB, W), jnp.float32)
    s, ar, ag, ab = z, z, z, z
    for k in range(K):
        aa = tab_ref[0, 0, 4 * k]
        cr = tab_ref[0, 0, 4 * k + 1]
        cg = tab_ref[0, 0, 4 * k + 2]
        cb = tab_ref[0, 0, 4 * k + 3]
        t = (aa + cr * r) + cg * g + cb * b
        e = jnp.exp(t)
        s = s + e
        ar = ar + e * cr
        ag = ag + e * cg
        ab = ab + e * cb
    inv = tab_ref[0, 0, 4 * K + 1] / s
    out_ref[0, 0] = ar * inv
    out_ref[0, 1] = ag * inv
    out_ref[0, 2] = ab * inv


_tc_quantize = pl.pallas_call(
    _tc_body,
    grid=(B, (H - HS_SC) // ROWB),
    in_specs=[
        pl.BlockSpec((1, C, ROWB, W),
                     lambda b, rb: (b, 0, HS_SC // ROWB + rb, 0)),
        pl.BlockSpec((1, 1, TAB_ROWS), lambda b, rb: (b, 0, 0),
                     memory_space=pltpu.SMEM),
    ],
    out_specs=pl.BlockSpec((1, C, ROWB, W),
                           lambda b, rb: (b, 0, HS_SC // ROWB + rb, 0)),
    out_shape=jax.ShapeDtypeStruct((B, C, H, W), jnp.float32),
)


@jax.jit
def kernel(images, palettes, temperature):
    planar = jnp.transpose(images, (0, 3, 1, 2))
    flat = planar.reshape(-1)
    g2 = (-1.0 / temperature).astype(jnp.float32)
    aa = g2 * jnp.sum(palettes * palettes, axis=-1, keepdims=True)
    cc = (-2.0 * g2) * palettes
    per_k = jnp.concatenate([aa, cc], axis=-1).reshape(B, 4 * K)
    extra = jnp.stack([jnp.broadcast_to(g2, (B,)),
                       jnp.broadcast_to(-0.5 / g2, (B,))], axis=1)
    pad = jnp.zeros((B, TAB_ROWS - 4 * K - 2), jnp.float32)
    tab = jnp.concatenate([per_k, extra, pad], axis=1)
    sc_flat = _sc_quantize(flat, tab.reshape(-1))
    tc_full = _tc_quantize(planar, tab[:, None, :])
    sc_part = sc_flat.reshape(B, C, HS_SC, W)
    full = lax.dynamic_update_slice(tc_full, sc_part, (0, 0, 0, 0))
    return jnp.transpose(full, (0, 2, 3, 1))
